# Initial kernel scaffold; baseline (speedup 1.0000x reference)
import jax
import jax.numpy as jnp
from jax.experimental import pallas as pl


def kernel(handler_parameters, handler_parameter_idx, parameter_delta):
    raise NotImplementedError("write your pallas kernel here")


# speedup vs baseline: 12.7619x; 12.7619x over previous
import functools

import jax
import jax.numpy as jnp
from jax import lax
from jax.experimental import pallas as pl
from jax.experimental.pallas import tpu as pltpu
from jax.experimental.pallas import tpu_sc as plsc

N_INTER = 2097152
N_COLS_ = 4
N_DELTA = 4096
N_FLAT = N_INTER * N_COLS_

NC = 2
NS = 16
NW = NC * NS
PER_W = N_FLAT // NW
CHUNK = 8192
NCHUNK = PER_W // CHUNK
LANES = 16
VECS = CHUNK // LANES

_mesh = plsc.VectorSubcoreMesh(core_axis_name="c", subcore_axis_name="s")


@functools.partial(
    pl.kernel,
    mesh=_mesh,
    out_type=jax.ShapeDtypeStruct((N_FLAT,), jnp.float32),
    compiler_params=pltpu.CompilerParams(needs_layout_passes=False),
    scratch_types=[
        pltpu.VMEM((N_DELTA,), jnp.float32),
        pltpu.VMEM((CHUNK,), jnp.int32),
        pltpu.VMEM((CHUNK,), jnp.float32),
    ],
)
def _sc_gather_add(hp_hbm, idx_hbm, delta_hbm, out_hbm, delta_v, idx_v, val_v):
    wid = lax.axis_index("s") * NC + lax.axis_index("c")
    base = wid * PER_W
    pltpu.sync_copy(delta_hbm, delta_v)

    def chunk_body(g, carry):
        off = base + g * CHUNK
        pltpu.sync_copy(idx_hbm.at[pl.ds(off, CHUNK)], idx_v)
        pltpu.sync_copy(hp_hbm.at[pl.ds(off, CHUNK)], val_v)

        def vec_body(i, c):
            s = pl.ds(i * LANES, LANES)
            gv = plsc.load_gather(delta_v, [idx_v[s]])
            val_v[s] = val_v[s] + gv
            return c

        lax.fori_loop(0, VECS, vec_body, 0, unroll=4)
        pltpu.sync_copy(val_v, out_hbm.at[pl.ds(off, CHUNK)])
        return carry

    lax.fori_loop(0, NCHUNK, chunk_body, 0)


def kernel(handler_parameters, handler_parameter_idx, parameter_delta):
    hp = handler_parameters.reshape(-1)
    idx = handler_parameter_idx.reshape(-1)
    out = _sc_gather_add(hp, idx, parameter_delta)
    return out.reshape(handler_parameters.shape)


# speedup vs baseline: 13.1576x; 1.0310x over previous
import functools

import jax
import jax.numpy as jnp
from jax import lax
from jax.experimental import pallas as pl
from jax.experimental.pallas import tpu as pltpu
from jax.experimental.pallas import tpu_sc as plsc

N_INTER = 2097152
N_COLS_ = 4
N_DELTA = 4096
N_FLAT = N_INTER * N_COLS_

NC = 2
NS = 16
NW = NC * NS
PER_W = N_FLAT // NW
CHUNK = 8192
NCHUNK = PER_W // CHUNK
LANES = 16
VECS = CHUNK // LANES
NBUF = 4
NGROUP = NCHUNK // NBUF

_mesh = plsc.VectorSubcoreMesh(core_axis_name="c", subcore_axis_name="s")


@functools.partial(
    pl.kernel,
    mesh=_mesh,
    out_type=jax.ShapeDtypeStruct((N_FLAT,), jnp.float32),
    compiler_params=pltpu.CompilerParams(needs_layout_passes=False),
    scratch_types=[
        pltpu.VMEM((N_DELTA,), jnp.float32),
        [pltpu.VMEM((CHUNK,), jnp.int32)] * NBUF,
        [pltpu.VMEM((CHUNK,), jnp.float32)] * NBUF,
        [pltpu.VMEM((CHUNK,), jnp.float32)] * NBUF,
        [pltpu.SemaphoreType.DMA] * NBUF,
        [pltpu.SemaphoreType.DMA] * NBUF,
    ],
)
def _sc_gather_add(
    hp_hbm, idx_hbm, delta_hbm, out_hbm, delta_v, idx_v, val_v, res_v,
    sems_in, sems_out,
):
    wid = lax.axis_index("s") * NC + lax.axis_index("c")
    base = wid * PER_W
    pltpu.sync_copy(delta_hbm, delta_v)

    def start_in(g, b):
        off = base + g * CHUNK
        pltpu.async_copy(idx_hbm.at[pl.ds(off, CHUNK)], idx_v[b], sems_in[b])
        pltpu.async_copy(hp_hbm.at[pl.ds(off, CHUNK)], val_v[b], sems_in[b])

    def wait_in(b):
        pltpu.make_async_copy(idx_hbm.at[pl.ds(base, CHUNK)], idx_v[b], sems_in[b]).wait()
        pltpu.make_async_copy(hp_hbm.at[pl.ds(base, CHUNK)], val_v[b], sems_in[b]).wait()

    def start_out(g, b):
        off = base + g * CHUNK
        pltpu.async_copy(res_v[b], out_hbm.at[pl.ds(off, CHUNK)], sems_out[b])

    def wait_out(b):
        pltpu.make_async_copy(res_v[b], out_hbm.at[pl.ds(base, CHUNK)], sems_out[b]).wait()

    for b in range(NBUF):
        start_in(b, b)

    def group_body(G, carry):
        g0 = G * NBUF
        for b in range(NBUF):
            g = g0 + b
            wait_in(b)

            @pl.when(G > 0)
            def _():
                wait_out(b)

            ib, vb, rb = idx_v[b], val_v[b], res_v[b]

            @plsc.parallel_loop(0, VECS, unroll=8)
            def vec_body(i):
                s = pl.ds(i * LANES, LANES)
                rb[s] = vb[s] + plsc.load_gather(delta_v, [ib[s]])

            start_out(g, b)

            @pl.when(g + NBUF < NCHUNK)
            def _():
                start_in(g + NBUF, b)
        return carry

    lax.fori_loop(0, NGROUP, group_body, 0)
    for b in range(NBUF):
        wait_out(b)


def kernel(handler_parameters, handler_parameter_idx, parameter_delta):
    hp = handler_parameters.reshape(-1)
    idx = handler_parameter_idx.reshape(-1)
    out = _sc_gather_add(hp, idx, parameter_delta)
    return out.reshape(handler_parameters.shape)
